# baseline (speedup 1.0000x reference)
import dataclasses

import jax
import jax.numpy as jnp
from jax import lax
from jax.experimental import pallas as pl
from jax.experimental.pallas import tpu as pltpu
from jax.experimental.pallas import tpu_sc as plsc

N = 10000
E = 320000
D = 128

NC = 2
NS = 16
K = 128
CH = 80
E_PAD = NC * NS * CH * K
N_PAD = 10240
STRIPE = N_PAD // NS
RB = 2048

_MESH = plsc.VectorSubcoreMesh(core_axis_name="c", subcore_axis_name="s")
_CP = pltpu.CompilerParams()
if "needs_layout_passes" in pltpu.CompilerParams.__dataclass_fields__:
    _CP = dataclasses.replace(_CP, needs_layout_passes=False)
_HIGH = lax.Precision.HIGHEST
_DN = (((1,), (0,)), ((), ()))


def _fill(ref, rows, width, value):
    vec = jnp.full((16,), value, jnp.float32)

    @pl.loop(0, rows)
    def _(r):
        for c in range(width // 16):
            ref.at[r, pl.ds(c * 16, 16)][...] = vec


def _deg_body(dst_hbm, out_hbm, dst_v, deg_l, buf_v, res_v, sh):
    ci = lax.axis_index("c")
    si = lax.axis_index("s")
    pltpu.sync_copy(dst_hbm.at[ci, si], dst_v)

    @pl.loop(0, N_PAD // 16)
    def _(i):
        deg_l.at[pl.ds(i * 16, 16)][...] = jnp.zeros((16,), jnp.float32)

    ones = jnp.ones((16,), jnp.float32)

    @pl.loop(0, CH)
    def _(r):
        for c in range(K // 16):
            idx = dst_v.at[r, pl.ds(c * 16, 16)][...]
            plsc.addupdate_scatter(deg_l, [idx], ones)

    pltpu.sync_copy(deg_l, sh.at[si])
    plsc.subcore_barrier()
    pltpu.sync_copy(sh.at[:, pl.ds(si * STRIPE, STRIPE)], buf_v)

    @pl.loop(0, STRIPE // 16)
    def _(cc):
        tot = buf_v.at[0, pl.ds(cc * 16, 16)][...]
        for r in range(1, NS):
            tot = tot + buf_v.at[r, pl.ds(cc * 16, 16)][...]
        res_v.at[pl.ds(cc * 16, 16)][...] = tot

    pltpu.sync_copy(res_v, out_hbm.at[ci, pl.ds(si * STRIPE, STRIPE)])


_sc_deg = pl.kernel(
    _deg_body,
    out_type=jax.ShapeDtypeStruct((NC, N_PAD), jnp.float32),
    mesh=_MESH,
    scratch_types=[
        pltpu.VMEM((CH, K), jnp.int32),
        pltpu.VMEM((N_PAD,), jnp.float32),
        pltpu.VMEM((NS, STRIPE), jnp.float32),
        pltpu.VMEM((STRIPE,), jnp.float32),
        pltpu.VMEM_SHARED((NS, N_PAD), jnp.float32),
    ],
    compiler_params=_CP,
)


def _scatter_body(g_hbm, src_hbm, dst_hbm, out_hbm, src_v, dst_v, rows_v, acc,
                  sem):
    ci = lax.axis_index("c")
    si = lax.axis_index("s")
    pltpu.sync_copy(src_hbm.at[ci, si], src_v)
    pltpu.sync_copy(dst_hbm.at[ci, si], dst_v)
    _fill(rows_v, K, D, 0.0)

    @pl.loop(0, STRIPE // K)
    def _(t):
        pltpu.sync_copy(rows_v, acc.at[pl.ds(si * STRIPE + t * K, K)])

    plsc.subcore_barrier()

    @pl.loop(0, CH)
    def _(j):
        pltpu.async_copy(g_hbm.at[src_v.at[j]], rows_v, sem).wait()
        pltpu.sync_copy(rows_v, acc.at[dst_v.at[j]], add=True)

    plsc.subcore_barrier()
    pltpu.sync_copy(acc.at[pl.ds(si * STRIPE, STRIPE)],
                    out_hbm.at[ci, pl.ds(si * STRIPE, STRIPE)])


_sc_scatter = pl.kernel(
    _scatter_body,
    out_type=jax.ShapeDtypeStruct((NC, N_PAD, D), jnp.float32),
    mesh=_MESH,
    scratch_types=[
        pltpu.VMEM((CH, K), jnp.int32),
        pltpu.VMEM((CH, K), jnp.int32),
        pltpu.VMEM((K, D), jnp.float32),
        pltpu.VMEM_SHARED((N_PAD, D), jnp.float32),
        pltpu.SemaphoreType.DMA,
    ],
)


_BSF = pl.BlockSpec((RB, D), lambda i: (i, 0))
_BS1 = pl.BlockSpec((RB, 1), lambda i: (i, 0))
_BSB = pl.BlockSpec((1, D), lambda i: (0, 0))
_BSW = pl.BlockSpec((D, D), lambda i: (0, 0))


def _mm_body(x_ref, w_ref, o_ref):
    o_ref[...] = lax.dot_general(x_ref[...], w_ref[...], _DN, precision=_HIGH,
                                 preferred_element_type=jnp.float32)


def _tc_matmul(x_pad, W):
    return pl.pallas_call(
        _mm_body,
        grid=(N_PAD // RB,),
        in_specs=[_BSF, _BSW],
        out_specs=_BSF,
        out_shape=jax.ShapeDtypeStruct((N_PAD, D), jnp.float32),
    )(x_pad, W)


def _scale_body(da_ref, db_ref, m_ref, dinv_ref, g_ref):
    deg = da_ref[...] + db_ref[...] + 1.0
    dinv = lax.rsqrt(deg)
    dinv_ref[...] = dinv
    g_ref[...] = m_ref[...] * dinv


def _tc_scale(deg_a, deg_b, m1):
    return pl.pallas_call(
        _scale_body,
        grid=(N_PAD // RB,),
        in_specs=[_BS1, _BS1, _BSF],
        out_specs=[_BS1, _BSF],
        out_shape=[jax.ShapeDtypeStruct((N_PAD, 1), jnp.float32),
                   jax.ShapeDtypeStruct((N_PAD, D), jnp.float32)],
    )(deg_a, deg_b, m1)


def _mid_body(pa_ref, pb_ref, g_ref, dinv_ref, b_ref, w_ref, o_ref):
    s = pa_ref[...] + pb_ref[...] + g_ref[...]
    h = jnp.maximum(s * dinv_ref[...] + b_ref[...], 0.0)
    o_ref[...] = lax.dot_general(h, w_ref[...], _DN, precision=_HIGH,
                                 preferred_element_type=jnp.float32
                                 ) * dinv_ref[...]


def _tc_mid(pa, pb, g1, dinv, b1, W2):
    return pl.pallas_call(
        _mid_body,
        grid=(N_PAD // RB,),
        in_specs=[_BSF, _BSF, _BSF, _BS1, _BSB, _BSW],
        out_specs=_BSF,
        out_shape=jax.ShapeDtypeStruct((N_PAD, D), jnp.float32),
    )(pa, pb, g1, dinv, b1, W2)


def _final_body(pa_ref, pb_ref, g_ref, dinv_ref, b_ref, o_ref):
    s = pa_ref[...] + pb_ref[...] + g_ref[...]
    o_ref[...] = jnp.maximum(s * dinv_ref[...] + b_ref[...], 0.0)


def _tc_final(pa, pb, g2, dinv, b2):
    return pl.pallas_call(
        _final_body,
        grid=(N_PAD // RB,),
        in_specs=[_BSF, _BSF, _BSF, _BS1, _BSB],
        out_specs=pl.BlockSpec((RB, D), lambda i: (i, 0)),
        out_shape=jax.ShapeDtypeStruct((N, D), jnp.float32),
    )(pa, pb, g2, dinv, b2)


@jax.jit
def kernel(x, edge_index, W1, b1, W2, b2):
    x_pad = jnp.pad(x, ((0, N_PAD - N), (0, 0)))
    pad = jnp.full((E_PAD - E,), N, jnp.int32)
    src4 = jnp.concatenate([edge_index[0], pad]).reshape(NC, NS, CH, K)
    dst4 = jnp.concatenate([edge_index[1], pad]).reshape(NC, NS, CH, K)
    b1r = b1.reshape(1, D)
    b2r = b2.reshape(1, D)

    degp = _sc_deg(dst4)
    m1 = _tc_matmul(x_pad, W1)
    dinv, g1 = _tc_scale(degp[0].reshape(N_PAD, 1), degp[1].reshape(N_PAD, 1), m1)
    p1 = _sc_scatter(g1, src4, dst4)
    g2 = _tc_mid(p1[0], p1[1], g1, dinv, b1r, W2)
    p2 = _sc_scatter(g2, src4, dst4)
    return _tc_final(p2[0], p2[1], g2, dinv, b2r)


# speedup vs baseline: 1.1122x; 1.1122x over previous
import dataclasses

import jax
import jax.numpy as jnp
from jax import lax
from jax.experimental import pallas as pl
from jax.experimental.pallas import tpu as pltpu
from jax.experimental.pallas import tpu_sc as plsc

N = 10000
E = 320000
D = 128

NC = 2
NS = 16
K = 128
CH = 80
E_PAD = NC * NS * CH * K
N_PAD = 10240
STRIPE = N_PAD // NS
RB = 2048

_MESH = plsc.VectorSubcoreMesh(core_axis_name="c", subcore_axis_name="s")
_CP = pltpu.CompilerParams()
if "needs_layout_passes" in pltpu.CompilerParams.__dataclass_fields__:
    _CP = dataclasses.replace(_CP, needs_layout_passes=False)
_HIGH = lax.Precision.HIGHEST
_DN = (((1,), (0,)), ((), ()))


def _fill(ref, rows, width, value):
    vec = jnp.full((16,), value, jnp.float32)

    @pl.loop(0, rows)
    def _(r):
        for c in range(width // 16):
            ref.at[r, pl.ds(c * 16, 16)][...] = vec


def _deg_body(dst_hbm, out_hbm, dst_v, buf_v, acc):
    ci = lax.axis_index("c")
    si = lax.axis_index("s")
    pltpu.sync_copy(dst_hbm.at[ci, si], dst_v)
    _fill(buf_v, K, D, 0.0)

    @pl.loop(0, STRIPE // K)
    def _(t):
        pltpu.sync_copy(buf_v, acc.at[pl.ds(si * STRIPE + t * K, K)])

    _fill(buf_v, K, D, 1.0)
    plsc.subcore_barrier()

    @pl.loop(0, CH)
    def _(j):
        pltpu.sync_copy(buf_v, acc.at[dst_v.at[j]], add=True)

    plsc.subcore_barrier()
    pltpu.sync_copy(acc.at[pl.ds(si * STRIPE, STRIPE)],
                    out_hbm.at[ci, pl.ds(si * STRIPE, STRIPE)])


_sc_deg = pl.kernel(
    _deg_body,
    out_type=jax.ShapeDtypeStruct((NC, N_PAD, D), jnp.float32),
    mesh=_MESH,
    scratch_types=[
        pltpu.VMEM((CH, K), jnp.int32),
        pltpu.VMEM((K, D), jnp.float32),
        pltpu.VMEM_SHARED((N_PAD, D), jnp.float32),
    ],
)


def _scatter_body(g_hbm, src_hbm, dst_hbm, out_hbm, src_v, dst_v, rows_v, acc,
                  sem):
    ci = lax.axis_index("c")
    si = lax.axis_index("s")
    pltpu.sync_copy(src_hbm.at[ci, si], src_v)
    pltpu.sync_copy(dst_hbm.at[ci, si], dst_v)
    _fill(rows_v, K, D, 0.0)

    @pl.loop(0, STRIPE // K)
    def _(t):
        pltpu.sync_copy(rows_v, acc.at[pl.ds(si * STRIPE + t * K, K)])

    plsc.subcore_barrier()

    @pl.loop(0, CH)
    def _(j):
        pltpu.async_copy(g_hbm.at[src_v.at[j]], rows_v, sem).wait()
        pltpu.sync_copy(rows_v, acc.at[dst_v.at[j]], add=True)

    plsc.subcore_barrier()
    pltpu.sync_copy(acc.at[pl.ds(si * STRIPE, STRIPE)],
                    out_hbm.at[ci, pl.ds(si * STRIPE, STRIPE)])


_sc_scatter = pl.kernel(
    _scatter_body,
    out_type=jax.ShapeDtypeStruct((NC, N_PAD, D), jnp.float32),
    mesh=_MESH,
    scratch_types=[
        pltpu.VMEM((CH, K), jnp.int32),
        pltpu.VMEM((CH, K), jnp.int32),
        pltpu.VMEM((K, D), jnp.float32),
        pltpu.VMEM_SHARED((N_PAD, D), jnp.float32),
        pltpu.SemaphoreType.DMA,
    ],
)


_BSF = pl.BlockSpec((RB, D), lambda i: (i, 0))
_BS1 = pl.BlockSpec((RB, 1), lambda i: (i, 0))
_BSB = pl.BlockSpec((1, D), lambda i: (0, 0))
_BSW = pl.BlockSpec((D, D), lambda i: (0, 0))


def _mm_body(x_ref, w_ref, o_ref):
    o_ref[...] = lax.dot_general(x_ref[...], w_ref[...], _DN, precision=_HIGH,
                                 preferred_element_type=jnp.float32)


def _tc_matmul(x_pad, W):
    return pl.pallas_call(
        _mm_body,
        grid=(N_PAD // RB,),
        in_specs=[_BSF, _BSW],
        out_specs=_BSF,
        out_shape=jax.ShapeDtypeStruct((N_PAD, D), jnp.float32),
    )(x_pad, W)


def _scale_body(da_ref, db_ref, m_ref, dinv_ref, g_ref):
    deg = da_ref[...][:, 0:1] + db_ref[...][:, 0:1] + 1.0
    dinv = lax.rsqrt(deg)
    dinv_ref[...] = dinv
    g_ref[...] = m_ref[...] * dinv


def _tc_scale(deg_a, deg_b, m1):
    return pl.pallas_call(
        _scale_body,
        grid=(N_PAD // RB,),
        in_specs=[_BSF, _BSF, _BSF],
        out_specs=[_BS1, _BSF],
        out_shape=[jax.ShapeDtypeStruct((N_PAD, 1), jnp.float32),
                   jax.ShapeDtypeStruct((N_PAD, D), jnp.float32)],
    )(deg_a, deg_b, m1)


def _mid_body(pa_ref, pb_ref, g_ref, dinv_ref, b_ref, w_ref, o_ref):
    s = pa_ref[...] + pb_ref[...] + g_ref[...]
    h = jnp.maximum(s * dinv_ref[...] + b_ref[...], 0.0)
    o_ref[...] = lax.dot_general(h, w_ref[...], _DN, precision=_HIGH,
                                 preferred_element_type=jnp.float32
                                 ) * dinv_ref[...]


def _tc_mid(pa, pb, g1, dinv, b1, W2):
    return pl.pallas_call(
        _mid_body,
        grid=(N_PAD // RB,),
        in_specs=[_BSF, _BSF, _BSF, _BS1, _BSB, _BSW],
        out_specs=_BSF,
        out_shape=jax.ShapeDtypeStruct((N_PAD, D), jnp.float32),
    )(pa, pb, g1, dinv, b1, W2)


def _final_body(pa_ref, pb_ref, g_ref, dinv_ref, b_ref, o_ref):
    s = pa_ref[...] + pb_ref[...] + g_ref[...]
    o_ref[...] = jnp.maximum(s * dinv_ref[...] + b_ref[...], 0.0)


def _tc_final(pa, pb, g2, dinv, b2):
    return pl.pallas_call(
        _final_body,
        grid=(N_PAD // RB,),
        in_specs=[_BSF, _BSF, _BSF, _BS1, _BSB],
        out_specs=pl.BlockSpec((RB, D), lambda i: (i, 0)),
        out_shape=jax.ShapeDtypeStruct((N, D), jnp.float32),
    )(pa, pb, g2, dinv, b2)


@jax.jit
def kernel(x, edge_index, W1, b1, W2, b2):
    x_pad = jnp.pad(x, ((0, N_PAD - N), (0, 0)))
    pad = jnp.full((E_PAD - E,), N, jnp.int32)
    src4 = jnp.concatenate([edge_index[0], pad]).reshape(NC, NS, CH, K)
    dst4 = jnp.concatenate([edge_index[1], pad]).reshape(NC, NS, CH, K)
    b1r = b1.reshape(1, D)
    b2r = b2.reshape(1, D)

    degp = _sc_deg(dst4)
    m1 = _tc_matmul(x_pad, W1)
    dinv, g1 = _tc_scale(degp[0], degp[1], m1)
    p1 = _sc_scatter(g1, src4, dst4)
    g2 = _tc_mid(p1[0], p1[1], g1, dinv, b1r, W2)
    p2 = _sc_scatter(g2, src4, dst4)
    return _tc_final(p2[0], p2[1], g2, dinv, b2r)


# speedup vs baseline: 1.6270x; 1.4630x over previous
import dataclasses

import jax
import jax.numpy as jnp
from jax import lax
from jax.experimental import pallas as pl
from jax.experimental.pallas import tpu as pltpu
from jax.experimental.pallas import tpu_sc as plsc

N = 10000
E = 320000
D = 128

NC = 2
NS = 16
K = 128
CH = 79
E_PAD = NC * NS * CH * K
N_PAD = 10240
STRIPE = N_PAD // NS
RB = 2048

_MESH = plsc.VectorSubcoreMesh(core_axis_name="c", subcore_axis_name="s")
_CP = pltpu.CompilerParams()
if "needs_layout_passes" in pltpu.CompilerParams.__dataclass_fields__:
    _CP = dataclasses.replace(_CP, needs_layout_passes=False)
_HIGH = lax.Precision.HIGHEST
_DN = (((1,), (0,)), ((), ()))


def _fill(ref, rows, width, value):
    vec = jnp.full((16,), value, jnp.float32)

    @pl.loop(0, rows)
    def _(r):
        for c in range(width // 16):
            ref.at[r, pl.ds(c * 16, 16)][...] = vec


def _deg_body(dst_hbm, out_hbm, dst_v, buf_v, acc):
    ci = lax.axis_index("c")
    si = lax.axis_index("s")
    pltpu.sync_copy(dst_hbm.at[ci, si], dst_v)
    _fill(buf_v, K, D, 0.0)

    @pl.loop(0, STRIPE // K)
    def _(t):
        pltpu.sync_copy(buf_v, acc.at[pl.ds(si * STRIPE + t * K, K)])

    _fill(buf_v, K, D, 1.0)
    plsc.subcore_barrier()

    @pl.loop(0, CH)
    def _(j):
        pltpu.sync_copy(buf_v, acc.at[dst_v.at[j]], add=True)

    plsc.subcore_barrier()
    pltpu.sync_copy(acc.at[pl.ds(si * STRIPE, STRIPE)],
                    out_hbm.at[ci, pl.ds(si * STRIPE, STRIPE)])


_sc_deg = pl.kernel(
    _deg_body,
    out_type=jax.ShapeDtypeStruct((NC, N_PAD, D), jnp.float32),
    mesh=_MESH,
    scratch_types=[
        pltpu.VMEM((CH, K), jnp.int32),
        pltpu.VMEM((K, D), jnp.float32),
        pltpu.VMEM_SHARED((N_PAD, D), jnp.float32),
    ],
)


def _scatter_body(g_hbm, src_hbm, dst_hbm, out_hbm, src_v, dst_v, rows_v, acc,
                  sem):
    ci = lax.axis_index("c")
    si = lax.axis_index("s")
    pltpu.sync_copy(src_hbm.at[ci, si], src_v)
    pltpu.sync_copy(dst_hbm.at[ci, si], dst_v)
    _fill(rows_v, K, D, 0.0)

    @pl.loop(0, STRIPE // K)
    def _(t):
        pltpu.sync_copy(rows_v, acc.at[pl.ds(si * STRIPE + t * K, K)])

    plsc.subcore_barrier()

    @pl.loop(0, CH)
    def _(j):
        pltpu.async_copy(g_hbm.at[src_v.at[j]], rows_v, sem).wait()
        pltpu.sync_copy(rows_v, acc.at[dst_v.at[j]], add=True)

    plsc.subcore_barrier()
    pltpu.sync_copy(acc.at[pl.ds(si * STRIPE, STRIPE)],
                    out_hbm.at[ci, pl.ds(si * STRIPE, STRIPE)])


_sc_scatter = pl.kernel(
    _scatter_body,
    out_type=jax.ShapeDtypeStruct((NC, N_PAD, D), jnp.float32),
    mesh=_MESH,
    scratch_types=[
        pltpu.VMEM((CH, K), jnp.int32),
        pltpu.VMEM((CH, K), jnp.int32),
        pltpu.VMEM((K, D), jnp.float32),
        pltpu.VMEM_SHARED((N_PAD, D), jnp.float32),
        pltpu.SemaphoreType.DMA,
    ],
)


_BSF = pl.BlockSpec((RB, D), lambda i: (i, 0))
_BS1 = pl.BlockSpec((RB, 1), lambda i: (i, 0))
_BSB = pl.BlockSpec((1, D), lambda i: (0, 0))
_BSW = pl.BlockSpec((D, D), lambda i: (0, 0))


def _mm_body(x_ref, w_ref, o_ref):
    o_ref[...] = lax.dot_general(x_ref[...], w_ref[...], _DN, precision=_HIGH,
                                 preferred_element_type=jnp.float32)


def _tc_matmul(x_pad, W):
    return pl.pallas_call(
        _mm_body,
        grid=(N_PAD // RB,),
        in_specs=[_BSF, _BSW],
        out_specs=_BSF,
        out_shape=jax.ShapeDtypeStruct((N_PAD, D), jnp.float32),
    )(x_pad, W)


def _scale_body(da_ref, db_ref, m_ref, dinv_ref, g_ref):
    deg = da_ref[...][:, 0:1] + db_ref[...][:, 0:1] + 1.0
    dinv = lax.rsqrt(deg)
    dinv_ref[...] = dinv
    g_ref[...] = m_ref[...] * dinv


def _tc_scale(deg_a, deg_b, m1):
    return pl.pallas_call(
        _scale_body,
        grid=(N_PAD // RB,),
        in_specs=[_BSF, _BSF, _BSF],
        out_specs=[_BS1, _BSF],
        out_shape=[jax.ShapeDtypeStruct((N_PAD, 1), jnp.float32),
                   jax.ShapeDtypeStruct((N_PAD, D), jnp.float32)],
    )(deg_a, deg_b, m1)


def _mid_body(pa_ref, pb_ref, g_ref, dinv_ref, b_ref, w_ref, o_ref):
    s = pa_ref[...] + pb_ref[...] + g_ref[...]
    h = jnp.maximum(s * dinv_ref[...] + b_ref[...], 0.0)
    o_ref[...] = lax.dot_general(h, w_ref[...], _DN, precision=_HIGH,
                                 preferred_element_type=jnp.float32
                                 ) * dinv_ref[...]


def _tc_mid(pa, pb, g1, dinv, b1, W2):
    return pl.pallas_call(
        _mid_body,
        grid=(N_PAD // RB,),
        in_specs=[_BSF, _BSF, _BSF, _BS1, _BSB, _BSW],
        out_specs=_BSF,
        out_shape=jax.ShapeDtypeStruct((N_PAD, D), jnp.float32),
    )(pa, pb, g1, dinv, b1, W2)


def _final_body(pa_ref, pb_ref, g_ref, dinv_ref, b_ref, o_ref):
    s = pa_ref[...] + pb_ref[...] + g_ref[...]
    o_ref[...] = jnp.maximum(s * dinv_ref[...] + b_ref[...], 0.0)


def _tc_final(pa, pb, g2, dinv, b2):
    return pl.pallas_call(
        _final_body,
        grid=(N_PAD // RB,),
        in_specs=[_BSF, _BSF, _BSF, _BS1, _BSB],
        out_specs=pl.BlockSpec((RB, D), lambda i: (i, 0)),
        out_shape=jax.ShapeDtypeStruct((N, D), jnp.float32),
    )(pa, pb, g2, dinv, b2)


@jax.jit
def kernel(x, edge_index, W1, b1, W2, b2):
    x_pad = jnp.pad(x, ((0, N_PAD - N), (0, 0)))
    pad = jnp.full((E_PAD - E,), N, jnp.int32)
    src4 = jnp.concatenate([edge_index[0], pad]).reshape(NC, NS, CH, K)
    dst4 = jnp.concatenate([edge_index[1], pad]).reshape(NC, NS, CH, K)
    b1r = b1.reshape(1, D)
    b2r = b2.reshape(1, D)

    degp = _sc_deg(dst4)
    m1 = _tc_matmul(x_pad, W1)
    dinv, g1 = _tc_scale(degp[0], degp[1], m1)
    p1 = _sc_scatter(g1, src4, dst4)
    g2 = _tc_mid(p1[0], p1[1], g1, dinv, b1r, W2)
    p2 = _sc_scatter(g2, src4, dst4)
    return _tc_final(p2[0], p2[1], g2, dinv, b2r)
